# baseline (speedup 1.0000x reference)
import functools

import jax
import jax.numpy as jnp
from jax import lax
from jax.experimental import pallas as pl
from jax.experimental.pallas import tpu as pltpu
from jax.experimental.pallas import tpu_sc as plsc

B = 32
IMG = 256
NPIX = IMG * IMG
WIN = 251
NPATCH = 15876
K = 36
L = 16
PER = 8064
NPT = 125
NBG = 4
NUNIT = NBG * NPT
KG = 12
SLABR = 10
SLABW = SLABR * IMG
TPV = NPATCH - 124 * 128


def _index_tables(mask):
    rows = (mask // WIN).astype(jnp.int32)
    cols = (mask % WIN).astype(jnp.int32)
    pidx = rows[:PER] * IMG + cols[:PER]
    k = jnp.arange(64, dtype=jnp.int32)
    koff = (k // 6) * IMG + k % 6
    pt = jnp.arange(144, dtype=jnp.int32)
    p0 = jnp.minimum(pt * 128, NPATCH - 1)
    rowb = jnp.minimum(rows[p0], IMG - SLABR)
    rowsrc = rowb * IMG
    offu = (pt // 63) * (128 * IMG) - rowsrc
    return pidx, koff.astype(jnp.int32), offu.astype(jnp.int32), rowsrc


_MESH = plsc.VectorSubcoreMesh(core_axis_name="c", subcore_axis_name="s")


@functools.partial(
    pl.kernel,
    mesh=_MESH,
    compiler_params=pltpu.CompilerParams(needs_layout_passes=False),
    out_type=jax.ShapeDtypeStruct((K, B, NPATCH), jnp.float32),
    scratch_types=[
        pltpu.VMEM((2, 8, SLABW), jnp.float32),
        pltpu.VMEM((PER,), jnp.int32),
        pltpu.VMEM((64,), jnp.int32),
        pltpu.VMEM((144,), jnp.int32),
        pltpu.VMEM((144,), jnp.int32),
        pltpu.VMEM((3, KG, 8, 128), jnp.float32),
        pltpu.SemaphoreType.DMA((2,)),
        pltpu.SemaphoreType.DMA((3,)),
    ],
)
def _unfold(x_hbm, pidx_hbm, koff_hbm, offu_hbm, rowsrc_hbm, out_hbm,
            slab, pidx_v, koff_v, offu_v, rowsrc_v, ring, slsems, rsems):
    wid = lax.axis_index("s") * 2 + lax.axis_index("c")
    pltpu.sync_copy(pidx_hbm, pidx_v)
    pltpu.sync_copy(koff_hbm, koff_v)
    pltpu.sync_copy(offu_hbm, offu_v)
    pltpu.sync_copy(rowsrc_hbm, rowsrc_v)

    def scalar_at(ref, i):
        return ref[pl.ds(i, L)][0]

    def load_slab(u, slot):
        bg = lax.rem(u, NBG)
        src0 = pl.multiple_of(scalar_at(rowsrc_v, u // NBG), IMG)
        pltpu.async_copy(
            x_hbm.at[pl.ds(8 * bg, 8), pl.ds(src0, SLABW)],
            slab.at[slot], slsems.at[slot])

    def wait_slab(slot):
        pltpu.make_async_copy(
            x_hbm.at[pl.ds(0, 8), pl.ds(0, SLABW)],
            slab.at[slot], slsems.at[slot]).wait()

    def gather_group(g, slot, off_u, phase0):
        slv = jnp.full((L,), slot, jnp.int32)
        bound = jnp.full((L,), SLABW - 1, jnp.int32)

        @plsc.parallel_loop(0, KG, unroll=1)
        def _per_k(kk):
            ksc = scalar_at(koff_v, g * KG + kk) + off_u
            spl = jnp.full((L,), ksc, jnp.int32)
            for bi in range(8):
                biv = jnp.full((L,), bi, jnp.int32)
                for q in range(8):
                    t = pidx_v[pl.ds(phase0 + q * L, L)] + spl
                    t = jnp.minimum(t, bound)
                    ring[g, kk, bi, pl.ds(q * L, L)] = (
                        plsc.load_gather(slab, [slv, biv, t]))

    def wait_ring_full(g):
        pltpu.make_async_copy(
            ring.at[g],
            out_hbm.at[pl.ds(g * KG, KG), pl.ds(0, 8), pl.ds(0, 128)],
            rsems.at[g]).wait()

    def do_unit(u, slot, first):
        wait_slab(slot)
        u2 = u + 32

        @pl.when(u2 < NUNIT)
        def _prefetch():
            load_slab(u2, 1 - slot)

        bg = lax.rem(u, NBG)
        pt = u // NBG
        off_u = scalar_at(offu_v, pt)
        phase0 = lax.rem(pt, 63) * 128
        for g in range(3):
            @pl.when(jnp.logical_not(first))
            def _wait(g=g):
                wait_ring_full(g)

            gather_group(g, slot, off_u, phase0)
            pltpu.async_copy(
                ring.at[g],
                out_hbm.at[pl.ds(g * KG, KG), pl.ds(8 * bg, 8),
                           pl.ds(128 * pt, 128)],
                rsems.at[g])

    load_slab(wid, 0)

    def body(j, carry):
        u = wid + 32 * j

        @pl.when(u < NUNIT)
        def _work():
            do_unit(u, lax.rem(j, 2), j == 0)

        return carry

    lax.fori_loop(0, 16, body, 0)

    for g in range(3):
        wait_ring_full(g)


def kernel(input_data, mask):
    x2 = input_data.reshape(B, NPIX)
    pidx, koff, offu, rowsrc = _index_tables(mask)
    out_phys = _unfold(x2, pidx, koff, offu, rowsrc)
    return jnp.transpose(out_phys, (1, 2, 0))


# speedup vs baseline: 1.0026x; 1.0026x over previous
import functools

import jax
import jax.numpy as jnp
from jax import lax
from jax.experimental import pallas as pl
from jax.experimental.pallas import tpu as pltpu
from jax.experimental.pallas import tpu_sc as plsc

B = 32
IMG = 256
NPIX = IMG * IMG
WIN = 251
NPATCH = 15876
K = 36
L = 16
PER = 8064
NPT = 125
NBG = 4
NUNIT = NBG * NPT
KG = 12
SLABR = 10
SLABW = SLABR * IMG
TPV = NPATCH - 124 * 128


def _index_tables(mask):
    rows = (mask // WIN).astype(jnp.int32)
    cols = (mask % WIN).astype(jnp.int32)
    pidx = rows[:PER] * IMG + cols[:PER]
    k = jnp.arange(64, dtype=jnp.int32)
    koff = (k // 6) * IMG + k % 6
    pt = jnp.arange(144, dtype=jnp.int32)
    p0 = jnp.minimum(pt * 128, NPATCH - 1)
    rowb = jnp.minimum(rows[p0], IMG - SLABR)
    rowsrc = rowb * IMG
    offu = (pt // 63) * (128 * IMG) - rowsrc
    return pidx, koff.astype(jnp.int32), offu.astype(jnp.int32), rowsrc


_MESH = plsc.VectorSubcoreMesh(core_axis_name="c", subcore_axis_name="s")


@functools.partial(
    pl.kernel,
    mesh=_MESH,
    compiler_params=pltpu.CompilerParams(needs_layout_passes=False),
    out_type=jax.ShapeDtypeStruct((K, B, NPATCH), jnp.float32),
    scratch_types=[
        pltpu.VMEM((2, 8, SLABW), jnp.float32),
        pltpu.VMEM((PER,), jnp.int32),
        pltpu.VMEM((64,), jnp.int32),
        pltpu.VMEM((144,), jnp.int32),
        pltpu.VMEM((144,), jnp.int32),
        pltpu.VMEM((3, KG, 8, 128), jnp.float32),
        pltpu.SemaphoreType.DMA((2,)),
        pltpu.SemaphoreType.DMA((3,)),
    ],
)
def _unfold(x_hbm, pidx_hbm, koff_hbm, offu_hbm, rowsrc_hbm, out_hbm,
            slab, pidx_v, koff_v, offu_v, rowsrc_v, ring, slsems, rsems):
    wid = lax.axis_index("s") * 2 + lax.axis_index("c")
    pltpu.sync_copy(pidx_hbm, pidx_v)
    pltpu.sync_copy(koff_hbm, koff_v)
    pltpu.sync_copy(offu_hbm, offu_v)
    pltpu.sync_copy(rowsrc_hbm, rowsrc_v)

    def scalar_at(ref, i):
        return ref[pl.ds(i, L)][0]

    def load_slab(u, slot):
        bg = lax.rem(u, NBG)
        src0 = pl.multiple_of(scalar_at(rowsrc_v, u // NBG), IMG)
        pltpu.async_copy(
            x_hbm.at[pl.ds(8 * bg, 8), pl.ds(src0, SLABW)],
            slab.at[slot], slsems.at[slot])

    def wait_slab(slot):
        pltpu.make_async_copy(
            x_hbm.at[pl.ds(0, 8), pl.ds(0, SLABW)],
            slab.at[slot], slsems.at[slot]).wait()

    slab2 = slab.reshape(16, SLABW)
    ring2 = ring.reshape(3, KG * 8, 128)

    def gather_group(g, slot, off_u, phase0):
        @plsc.parallel_loop(0, KG, unroll=1)
        def _per_k(kk):
            ksc = scalar_at(koff_v, g * KG + kk) + off_u
            for bi in range(8):
                row = slot * 8 + bi
                rowv = jnp.full((L,), row, jnp.int32)
                spl = jnp.full((L,), ksc, jnp.int32)
                bound = jnp.full((L,), SLABW - 1, jnp.int32)
                for q in range(8):
                    t = pidx_v[pl.ds(phase0 + q * L, L)] + spl
                    t = jnp.minimum(t, bound)
                    ring2[g, kk * 8 + bi, pl.ds(q * L, L)] = (
                        plsc.load_gather(slab2, [rowv, t]))

    def wait_ring_full(g):
        pltpu.make_async_copy(
            ring.at[g],
            out_hbm.at[pl.ds(g * KG, KG), pl.ds(0, 8), pl.ds(0, 128)],
            rsems.at[g]).wait()

    def do_unit(u, slot, first):
        wait_slab(slot)
        u2 = u + 32

        @pl.when(u2 < NUNIT)
        def _prefetch():
            load_slab(u2, 1 - slot)

        bg = lax.rem(u, NBG)
        pt = u // NBG
        off_u = scalar_at(offu_v, pt)
        phase0 = lax.rem(pt, 63) * 128
        for g in range(3):
            @pl.when(jnp.logical_not(first))
            def _wait(g=g):
                wait_ring_full(g)

            gather_group(g, slot, off_u, phase0)
            pltpu.async_copy(
                ring.at[g],
                out_hbm.at[pl.ds(g * KG, KG), pl.ds(8 * bg, 8),
                           pl.ds(128 * pt, 128)],
                rsems.at[g])

    load_slab(wid, 0)

    def body(j, carry):
        u = wid + 32 * j

        @pl.when(u < NUNIT)
        def _work():
            do_unit(u, lax.rem(j, 2), j == 0)

        return carry

    lax.fori_loop(0, 16, body, 0)

    for g in range(3):
        wait_ring_full(g)


def kernel(input_data, mask):
    x2 = input_data.reshape(B, NPIX)
    pidx, koff, offu, rowsrc = _index_tables(mask)
    out_phys = _unfold(x2, pidx, koff, offu, rowsrc)
    return jnp.transpose(out_phys, (1, 2, 0))


# speedup vs baseline: 2.6294x; 2.6225x over previous
import functools

import jax
import jax.numpy as jnp
from jax import lax
from jax.experimental import pallas as pl
from jax.experimental.pallas import tpu as pltpu
from jax.experimental.pallas import tpu_sc as plsc

B = 32
IMG = 256
NPIX = IMG * IMG
WIN = 251
NPATCH = 15876
K = 36
L = 16
PER = 8064
NPT = 125
NBG = 4
NUNIT = NBG * NPT
KG = 12
SLABR = 10
SLABW = SLABR * IMG
TPV = NPATCH - 124 * 128


def _index_tables(mask):
    rows = (mask // WIN).astype(jnp.int32)
    cols = (mask % WIN).astype(jnp.int32)
    pidx = rows[:PER] * IMG + cols[:PER]
    k = jnp.arange(64, dtype=jnp.int32)
    koff = (k // 6) * IMG + k % 6
    pt = jnp.arange(144, dtype=jnp.int32)
    p0 = jnp.minimum(pt * 128, NPATCH - 1)
    rowb = jnp.minimum(rows[p0], IMG - SLABR)
    rowsrc = rowb * IMG
    offu = (pt // 63) * (128 * IMG) - rowsrc
    return pidx, koff.astype(jnp.int32), offu.astype(jnp.int32), rowsrc


_MESH = plsc.VectorSubcoreMesh(core_axis_name="c", subcore_axis_name="s")


@functools.partial(
    pl.kernel,
    mesh=_MESH,
    compiler_params=pltpu.CompilerParams(needs_layout_passes=False),
    out_type=jax.ShapeDtypeStruct((K, B, NPATCH), jnp.float32),
    scratch_types=[
        pltpu.VMEM((2 * 8 * SLABW,), jnp.float32),
        pltpu.VMEM((PER,), jnp.int32),
        pltpu.VMEM((64,), jnp.int32),
        pltpu.VMEM((144,), jnp.int32),
        pltpu.VMEM((144,), jnp.int32),
        pltpu.VMEM((3, KG, 8, 128), jnp.float32),
        pltpu.SemaphoreType.DMA((2,)),
        pltpu.SemaphoreType.DMA((3,)),
    ],
)
def _unfold(x_hbm, pidx_hbm, koff_hbm, offu_hbm, rowsrc_hbm, out_hbm,
            slab, pidx_v, koff_v, offu_v, rowsrc_v, ring, slsems, rsems):
    wid = lax.axis_index("s") * 2 + lax.axis_index("c")
    pltpu.sync_copy(pidx_hbm, pidx_v)
    pltpu.sync_copy(koff_hbm, koff_v)
    pltpu.sync_copy(offu_hbm, offu_v)
    pltpu.sync_copy(rowsrc_hbm, rowsrc_v)

    def scalar_at(ref, i):
        return ref[pl.ds(i, L)][0]

    def load_slab(u, slot):
        bg = lax.rem(u, NBG)
        src0 = pl.multiple_of(scalar_at(rowsrc_v, u // NBG), IMG)
        for bi in range(8):
            pltpu.async_copy(
                x_hbm.at[pl.ds((8 * bg + bi) * NPIX + src0, SLABW)],
                slab.at[pl.ds((slot * 8 + bi) * SLABW, SLABW)],
                slsems.at[slot])

    def wait_slab(slot):
        for bi in range(8):
            pltpu.make_async_copy(
                x_hbm.at[pl.ds(bi * SLABW, SLABW)],
                slab.at[pl.ds(bi * SLABW, SLABW)],
                slsems.at[slot]).wait()

    ring2 = ring.reshape(3, KG * 8, 128)

    def gather_group(g, slot, off_u, phase0):
        pvecs = [pidx_v[pl.ds(phase0 + q * L, L)] for q in range(8)]
        bounds = [
            jnp.full((L,), (slot * 8 + bi + 1) * SLABW - 1, jnp.int32)
            for bi in range(8)
        ]

        @plsc.parallel_loop(0, KG, unroll=1)
        def _per_k(kk):
            ksc = scalar_at(koff_v, g * KG + kk) + off_u
            for bi in range(8):
                spl = jnp.full((L,), ksc + (slot * 8 + bi) * SLABW,
                               jnp.int32)
                for q in range(8):
                    t = jnp.minimum(pvecs[q] + spl, bounds[bi])
                    ring2[g, kk * 8 + bi, pl.ds(q * L, L)] = (
                        plsc.load_gather(slab, [t]))

    def wait_ring_full(g):
        pltpu.make_async_copy(
            ring.at[g],
            out_hbm.at[pl.ds(g * KG, KG), pl.ds(0, 8), pl.ds(0, 128)],
            rsems.at[g]).wait()

    def do_unit(u, slot, first):
        wait_slab(slot)
        u2 = u + 32

        @pl.when(u2 < NUNIT)
        def _prefetch():
            load_slab(u2, 1 - slot)

        bg = lax.rem(u, NBG)
        pt = u // NBG
        off_u = scalar_at(offu_v, pt)
        phase0 = lax.rem(pt, 63) * 128
        for g in range(3):
            @pl.when(jnp.logical_not(first))
            def _wait(g=g):
                wait_ring_full(g)

            gather_group(g, slot, off_u, phase0)
            pltpu.async_copy(
                ring.at[g],
                out_hbm.at[pl.ds(g * KG, KG), pl.ds(8 * bg, 8),
                           pl.ds(128 * pt, 128)],
                rsems.at[g])

    load_slab(wid, 0)

    def body(j, carry):
        u = wid + 32 * j

        @pl.when(u < NUNIT)
        def _work():
            do_unit(u, lax.rem(j, 2), j == 0)

        return carry

    lax.fori_loop(0, 16, body, 0)

    for g in range(3):
        wait_ring_full(g)


def kernel(input_data, mask):
    x2 = input_data.reshape(B * NPIX)
    pidx, koff, offu, rowsrc = _index_tables(mask)
    out_phys = _unfold(x2, pidx, koff, offu, rowsrc)
    return jnp.transpose(out_phys, (1, 2, 0))


# speedup vs baseline: 2.6761x; 1.0178x over previous
import functools

import jax
import jax.numpy as jnp
from jax import lax
from jax.experimental import pallas as pl
from jax.experimental.pallas import tpu as pltpu
from jax.experimental.pallas import tpu_sc as plsc

B = 32
IMG = 256
NPIX = IMG * IMG
WIN = 251
NPATCH = 15876
K = 36
L = 16
PER = 8064
NPT = 125
NBG = 4
NUNIT = NBG * NPT
KG = 12
SLABR = 10
SLABW = SLABR * IMG


def _index_tables(mask):
    rows = (mask // WIN).astype(jnp.int32)
    cols = (mask % WIN).astype(jnp.int32)
    pidx = rows[:PER] * IMG + cols[:PER]
    k = jnp.arange(64, dtype=jnp.int32)
    koff = (k // 6) * IMG + k % 6
    pt = jnp.arange(144, dtype=jnp.int32)
    p0 = jnp.minimum(pt * 128, NPATCH - 1)
    rowb = jnp.minimum(rows[p0], IMG - SLABR)
    rowsrc = rowb * IMG
    offu = (pt // 63) * (128 * IMG) - rowsrc
    return pidx, koff.astype(jnp.int32), offu.astype(jnp.int32), rowsrc


_MESH = plsc.VectorSubcoreMesh(core_axis_name="c", subcore_axis_name="s")


@functools.partial(
    pl.kernel,
    mesh=_MESH,
    compiler_params=pltpu.CompilerParams(needs_layout_passes=False),
    out_type=jax.ShapeDtypeStruct((K, B, NPATCH), jnp.float32),
    scratch_types=[
        pltpu.VMEM((2 * 8 * SLABW,), jnp.float32),
        pltpu.VMEM((PER,), jnp.int32),
        pltpu.VMEM((64,), jnp.int32),
        pltpu.VMEM((144,), jnp.int32),
        pltpu.VMEM((144,), jnp.int32),
        pltpu.VMEM((3, KG, 8, 128), jnp.float32),
        pltpu.SemaphoreType.DMA((2,)),
        pltpu.SemaphoreType.DMA((3,)),
    ],
)
def _unfold(x_hbm, pidx_hbm, koff_hbm, offu_hbm, rowsrc_hbm, out_hbm,
            slab, pidx_v, koff_v, offu_v, rowsrc_v, ring, slsems, rsems):
    wid = lax.axis_index("s") * 2 + lax.axis_index("c")
    pltpu.sync_copy(pidx_hbm, pidx_v)
    pltpu.sync_copy(koff_hbm, koff_v)
    pltpu.sync_copy(offu_hbm, offu_v)
    pltpu.sync_copy(rowsrc_hbm, rowsrc_v)

    def scalar_at(ref, i):
        return ref[pl.ds(i, L)][0]

    def load_slab(u, slot):
        bg = lax.rem(u, NBG)
        src0 = pl.multiple_of(scalar_at(rowsrc_v, u // NBG), IMG)
        for bi in range(8):
            pltpu.async_copy(
                x_hbm.at[pl.ds((8 * bg + bi) * NPIX + src0, SLABW)],
                slab.at[pl.ds((slot * 8 + bi) * SLABW, SLABW)],
                slsems.at[slot])

    def wait_slab(slot):
        for bi in range(8):
            pltpu.make_async_copy(
                x_hbm.at[pl.ds(bi * SLABW, SLABW)],
                slab.at[pl.ds(bi * SLABW, SLABW)],
                slsems.at[slot]).wait()

    ring2 = ring.reshape(3, KG * 8, 128)

    def gather_group(g, slot, off_u, phase0):
        pvecs = [pidx_v[pl.ds(phase0 + q * L, L)] for q in range(8)]
        bounds = [
            jnp.full((L,), (slot * 8 + bi + 1) * SLABW - 1, jnp.int32)
            for bi in range(8)
        ]

        @plsc.parallel_loop(0, KG, unroll=1)
        def _per_k(kk):
            ksc = scalar_at(koff_v, g * KG + kk) + off_u
            for bi in range(8):
                spl = jnp.full((L,), ksc + (slot * 8 + bi) * SLABW,
                               jnp.int32)
                for q in range(8):
                    t = jnp.minimum(pvecs[q] + spl, bounds[bi])
                    ring2[g, kk * 8 + bi, pl.ds(q * L, L)] = (
                        plsc.load_gather(slab, [t]))

    def wait_ring_full(g):
        pltpu.make_async_copy(
            ring.at[g],
            out_hbm.at[pl.ds(g * KG, KG), pl.ds(0, 8), pl.ds(0, 128)],
            rsems.at[g]).wait()

    def do_unit(u, slot, first):
        wait_slab(slot)
        u2 = u + 32

        @pl.when(u2 < NUNIT)
        def _prefetch():
            load_slab(u2, 1 - slot)

        bg = lax.rem(u, NBG)
        pt = u // NBG
        off_u = scalar_at(offu_v, pt)
        phase0 = lax.rem(pt, 63) * 128
        for g in range(3):
            @pl.when(jnp.logical_not(first))
            def _wait(g=g):
                wait_ring_full(g)

            gather_group(g, slot, off_u, phase0)
            pltpu.async_copy(
                ring.at[g],
                out_hbm.at[pl.ds(g * KG, KG), pl.ds(8 * bg, 8),
                           pl.ds(128 * pt, 128)],
                rsems.at[g])

    load_slab(wid, 0)

    def body(j, carry):
        u = wid + 32 * j

        @pl.when(u < NUNIT)
        def _work():
            do_unit(u, lax.rem(j, 2), j == 0)

        return carry

    lax.fori_loop(0, 16, body, 0)

    for g in range(3):
        wait_ring_full(g)


def kernel(input_data, mask):
    x2 = input_data.reshape(B * NPIX)
    pidx, koff, offu, rowsrc = _index_tables(mask)
    out_phys = _unfold(x2, pidx, koff, offu, rowsrc)
    return jnp.transpose(out_phys, (1, 2, 0))
